# baseline (speedup 1.0000x reference)
import functools

import jax
import jax.numpy as jnp
from jax import lax
from jax.experimental import pallas as pl
from jax.experimental.pallas import tpu as pltpu
from jax.experimental.pallas import tpu_sc as plsc

_N_FIELDS = 26
_VOCAB = 100000
_EMBED = 32
_BATCH = 4096
_FLAT = _BATCH * _N_FIELDS
_CHUNK = 128


def _sc_gather(feat3d, tbl128):
    info = plsc.get_sparse_core_info()
    nc, ns = info.num_cores, info.num_subcores
    nw = nc * ns
    per_tile = _FLAT // nw
    chunks = per_tile // _CHUNK

    mesh = plsc.VectorSubcoreMesh(core_axis_name="c", subcore_axis_name="s")

    @functools.partial(
        pl.kernel,
        mesh=mesh,
        out_type=jax.ShapeDtypeStruct((_FLAT, _EMBED), jnp.float32),
        scratch_types=[
            pltpu.VMEM((chunks, _CHUNK), jnp.int32),
            pltpu.VMEM((chunks, _CHUNK), jnp.int32),
            pltpu.VMEM((2, _CHUNK, 128), jnp.float32),
            pltpu.VMEM((2, _CHUNK, _EMBED), jnp.float32),
            pltpu.SemaphoreType.DMA,
            pltpu.SemaphoreType.DMA,
        ],
        compiler_params=pltpu.CompilerParams(use_tc_tiling_on_sc=True,
                                             needs_layout_passes=False),
    )
    def gather_kernel(feat_hbm, tbl_hbm, out_hbm, ridx_v, qoff_v, raw_v,
                      outb_v, gsem, osem):
        wid = lax.axis_index("s") * nc + lax.axis_index("c")
        base = wid * per_tile
        pltpu.sync_copy(feat_hbm.at[wid], ridx_v)

        def chunk_body(j, _):
            def vec_body(k, _):
                v = ridx_v[j, pl.ds(k * 16, 16)]
                v = jnp.clip(v, 0, _VOCAB - 1)
                pos = j * _CHUNK + k * 16 + lax.iota(jnp.int32, 16)
                flat = v + (pos % _N_FIELDS) * _VOCAB
                ridx_v[j, pl.ds(k * 16, 16)] = flat >> 2
                qoff_v[j, pl.ds(k * 16, 16)] = (flat & 3) * _EMBED
                return 0
            return lax.fori_loop(0, _CHUNK // 16, vec_body, 0)

        lax.fori_loop(0, chunks, chunk_body, 0)

        iota = lax.iota(jnp.int32, 16)

        def extract_chunk(j, buf):
            def ext_body(t, _):
                r = t // 2
                doff = (t % 2) * 16
                rvec = jnp.full((16,), r, jnp.int32)
                q = plsc.load_gather(qoff_v, [jnp.full((16,), j, jnp.int32),
                                              rvec])
                lane = q + doff + iota
                vals = plsc.load_gather(
                    raw_v, [jnp.full((16,), buf, jnp.int32), rvec, lane])
                outb_v[buf, r, pl.ds(doff, 16)] = vals
                return 0
            lax.fori_loop(0, _CHUNK * 2, ext_body, 0)

        gathers = [None] * chunks
        outs = [None] * chunks
        gathers[0] = pltpu.async_copy(tbl_hbm.at[ridx_v.at[0]],
                                      raw_v.at[0], gsem)
        for j in range(chunks):
            if j + 1 < chunks:
                gathers[j + 1] = pltpu.async_copy(
                    tbl_hbm.at[ridx_v.at[j + 1]], raw_v.at[(j + 1) % 2], gsem)
            gathers[j].wait()
            if j >= 2:
                outs[j - 2].wait()
            extract_chunk(j, j % 2)
            outs[j] = pltpu.async_copy(
                outb_v.at[j % 2],
                out_hbm.at[pl.ds(base + j * _CHUNK, _CHUNK)], osem)
        outs[chunks - 2].wait()
        outs[chunks - 1].wait()

    return gather_kernel(feat3d, tbl128)


_VCHUNK = 512
_NFULL = _VOCAB // _VCHUNK
_VTAIL = _VOCAB - _NFULL * _VCHUNK
_FULL_STEPS = _N_FIELDS * _NFULL


def _sc_transpose(tv2, tail):
    info = plsc.get_sparse_core_info()
    nc = info.num_cores
    nw = nc * info.num_subcores
    per_tile = (_FULL_STEPS + nw - 1) // nw
    rows_full = _VCHUNK // 4
    rows_tail = _VTAIL // 4
    vpf = _VOCAB // 4

    mesh = plsc.VectorSubcoreMesh(core_axis_name="c", subcore_axis_name="s")

    @functools.partial(
        pl.kernel,
        mesh=mesh,
        out_type=jax.ShapeDtypeStruct((_N_FIELDS * vpf, 128), jnp.float32),
        scratch_types=[
            pltpu.VMEM((2, _EMBED, _VCHUNK), jnp.float32),
            pltpu.VMEM((2, rows_full, 128), jnp.float32),
            pltpu.SemaphoreType.DMA,
            pltpu.SemaphoreType.DMA,
            pltpu.SemaphoreType.DMA,
            pltpu.SemaphoreType.DMA,
        ],
        compiler_params=pltpu.CompilerParams(use_tc_tiling_on_sc=True,
                                             needs_layout_passes=False),
    )
    def transpose_kernel(tv_hbm, tail_hbm, out_hbm, in_v, outb_v, isem0,
                         isem1, osem0, osem1):
        isems = (isem0, isem1)
        osems = (osem0, osem1)
        wid = lax.axis_index("s") * nc + lax.axis_index("c")
        lo = wid * per_tile
        hi = jnp.minimum(lo + per_tile, _FULL_STEPS)
        iota = lax.iota(jnp.int32, 16)

        def fetch(g, buf):
            f = g // _NFULL
            s = g % _NFULL
            pltpu.async_copy(
                tv_hbm.at[pl.ds(f * _EMBED, _EMBED),
                          pl.ds(s * _VCHUNK, _VCHUNK)],
                in_v.at[buf], isems[buf])

        def wait_in(buf):
            pltpu.make_async_copy(
                tv_hbm.at[pl.ds(0, _EMBED), pl.ds(0, _VCHUNK)],
                in_v.at[buf], isems[buf]).wait()

        def relane(buf, nrows):
            def row_body(i, _):
                for k in range(8):
                    d_vec = (k % 2) * 16 + iota
                    lane = jnp.full((16,), i * 4 + (k >> 1), jnp.int32)
                    vals = plsc.load_gather(
                        in_v, [jnp.full((16,), buf, jnp.int32), d_vec, lane])
                    outb_v[buf, i, pl.ds(k * 16, 16)] = vals
                return 0
            lax.fori_loop(0, nrows, row_body, 0)

        def writeout(g, buf):
            f = g // _NFULL
            s = g % _NFULL
            base = f * vpf + s * rows_full
            pltpu.async_copy(outb_v.at[buf],
                             out_hbm.at[pl.ds(base, rows_full)], osems[buf])

        def wait_out(buf):
            pltpu.make_async_copy(outb_v.at[buf],
                                  out_hbm.at[pl.ds(0, rows_full)],
                                  osems[buf]).wait()

        npairs = (per_tile + 1) // 2

        @pl.when(lo < hi)
        def _():
            fetch(lo, 0)

            def pair_body(p, _):
                g0 = lo + p * 2

                @pl.when(g0 < hi)
                def _():
                    @pl.when(g0 + 1 < hi)
                    def _():
                        fetch(g0 + 1, 1)
                    wait_in(0)

                    @pl.when(p >= 1)
                    def _():
                        wait_out(0)
                    relane(0, rows_full)
                    writeout(g0, 0)

                g1 = g0 + 1

                @pl.when(g1 < hi)
                def _():
                    @pl.when(g1 + 1 < hi)
                    def _():
                        fetch(g1 + 1, 0)
                    wait_in(1)

                    @pl.when(p >= 1)
                    def _():
                        wait_out(1)
                    relane(1, rows_full)
                    writeout(g1, 1)
                return 0

            lax.fori_loop(0, npairs, pair_body, 0)
            wait_out(0)

            @pl.when(hi - lo >= 2)
            def _():
                wait_out(1)

        @pl.when(wid < _N_FIELDS)
        def _():
            pltpu.sync_copy(
                tv_hbm.at[pl.ds(wid * _EMBED, _EMBED),
                          pl.ds(_NFULL * _VCHUNK, 128)],
                in_v.at[0, :, pl.ds(0, 128)])
            relane(0, 32)
            pltpu.sync_copy(
                outb_v.at[0, pl.ds(0, 32)],
                out_hbm.at[pl.ds(wid * vpf + _NFULL * rows_full, 32)])
            pltpu.sync_copy(tail_hbm.at[wid], outb_v.at[1, pl.ds(0, 8)])
            pltpu.sync_copy(
                outb_v.at[1, pl.ds(0, 8)],
                out_hbm.at[pl.ds(wid * vpf + _NFULL * rows_full + 32, 8)])

    return transpose_kernel(tv2, tail)


def _tc_mlp(x, gamma, beta, w1, b1, w2, b2, w3, b3):
    def body(x_ref, g_ref, be_ref, w1_ref, b1_ref, w2_ref, b2_ref, w3_ref,
             b3_ref, o_ref):
        xv = x_ref[...]
        inv_n = 1.0 / xv.shape[0]
        mean = jnp.sum(xv, axis=0, keepdims=True) * inv_n
        ex2 = jnp.sum(xv * xv, axis=0, keepdims=True) * inv_n
        var = ex2 - mean * mean
        scale = g_ref[...] * lax.rsqrt(var + 1e-5)
        shift = be_ref[...] - mean * scale
        xn = xv * scale + shift
        h = lax.dot_general(xn, w1_ref[...], (((1,), (1,)), ((), ())),
                            preferred_element_type=jnp.float32)
        h = jnp.maximum(h + b1_ref[...], 0.0)
        h = lax.dot_general(h, w2_ref[...], (((1,), (1,)), ((), ())),
                            preferred_element_type=jnp.float32)
        h = jnp.maximum(h + b2_ref[...], 0.0)
        logits = lax.dot_general(h, w3_ref[...], (((1,), (1,)), ((), ())),
                                 preferred_element_type=jnp.float32)
        o_ref[...] = jax.nn.sigmoid(logits + b3_ref[0])

    n_in = 9
    w3_pad = jnp.zeros((8, w3.shape[1]), w3.dtype).at[0].set(w3[0])
    out = pl.pallas_call(
        body,
        out_shape=jax.ShapeDtypeStruct((_BATCH, 8), jnp.float32),
        in_specs=[
            pl.BlockSpec(memory_space=pltpu.SMEM) if i == n_in - 1
            else pl.BlockSpec(memory_space=pltpu.VMEM)
            for i in range(n_in)
        ],
    )(x, gamma.reshape(1, -1), beta.reshape(1, -1), w1, b1.reshape(1, -1),
      w2, b2.reshape(1, -1), w3_pad, b3)
    return out[:, 0]


def kernel(features, tables, gamma, beta, w1, b1, w2, b2, w3, b3):
    feat3d = features.reshape(32, _N_FIELDS, _CHUNK)
    tv2 = jnp.transpose(tables, (0, 2, 1)).reshape(_N_FIELDS * _EMBED, _VOCAB)
    tail = tables[:, _VOCAB - 32:, :].reshape(_N_FIELDS, 8, 128)
    tbl128 = _sc_transpose(tv2, tail)
    rows = _sc_gather(feat3d, tbl128)
    x = rows.reshape(_BATCH, _N_FIELDS * _EMBED)
    out = _tc_mlp(x, gamma, beta, w1, b1, w2, b2, w3, b3)
    return out.reshape(_BATCH)


# speedup vs baseline: 1.4029x; 1.4029x over previous
import functools

import jax
import jax.numpy as jnp
from jax import lax
from jax.experimental import pallas as pl
from jax.experimental.pallas import tpu as pltpu
from jax.experimental.pallas import tpu_sc as plsc

_N_FIELDS = 26
_VOCAB = 100000
_EMBED = 32
_BATCH = 4096
_FLAT = _BATCH * _N_FIELDS
_CHUNK = 128


def _sc_gather(feat3d, tbl128):
    info = plsc.get_sparse_core_info()
    nc, ns = info.num_cores, info.num_subcores
    nw = nc * ns
    per_tile = _FLAT // nw
    chunks = per_tile // _CHUNK

    mesh = plsc.VectorSubcoreMesh(core_axis_name="c", subcore_axis_name="s")

    @functools.partial(
        pl.kernel,
        mesh=mesh,
        out_type=jax.ShapeDtypeStruct((_FLAT, _EMBED), jnp.float32),
        scratch_types=[
            pltpu.VMEM((chunks, _CHUNK), jnp.int32),
            pltpu.VMEM((chunks, _CHUNK), jnp.int32),
            pltpu.VMEM((2, _CHUNK, 128), jnp.float32),
            pltpu.VMEM((2, _CHUNK, _EMBED), jnp.float32),
            pltpu.SemaphoreType.DMA,
            pltpu.SemaphoreType.DMA,
        ],
        compiler_params=pltpu.CompilerParams(use_tc_tiling_on_sc=True,
                                             needs_layout_passes=False),
    )
    def gather_kernel(feat_hbm, tbl_hbm, out_hbm, ridx_v, qoff_v, raw_v,
                      outb_v, gsem, osem):
        wid = lax.axis_index("s") * nc + lax.axis_index("c")
        base = wid * per_tile
        pltpu.sync_copy(feat_hbm.at[wid], ridx_v)

        def chunk_body(j, _):
            def vec_body(k, _):
                v = ridx_v[j, pl.ds(k * 16, 16)]
                v = jnp.clip(v, 0, _VOCAB - 1)
                pos = j * _CHUNK + k * 16 + lax.iota(jnp.int32, 16)
                flat = v + (pos % _N_FIELDS) * _VOCAB
                ridx_v[j, pl.ds(k * 16, 16)] = flat >> 2
                qoff_v[j, pl.ds(k * 16, 16)] = (flat & 3) * _EMBED
                return 0
            return lax.fori_loop(0, _CHUNK // 16, vec_body, 0)

        lax.fori_loop(0, chunks, chunk_body, 0)

        iota = lax.iota(jnp.int32, 16)

        def extract_chunk(j, buf):
            def ext_body(t, _):
                r = t // 2
                doff = (t % 2) * 16
                rvec = jnp.full((16,), r, jnp.int32)
                q = plsc.load_gather(qoff_v, [jnp.full((16,), j, jnp.int32),
                                              rvec])
                lane = q + doff + iota
                vals = plsc.load_gather(
                    raw_v, [jnp.full((16,), buf, jnp.int32), rvec, lane])
                outb_v[buf, r, pl.ds(doff, 16)] = vals
                return 0
            lax.fori_loop(0, _CHUNK * 2, ext_body, 0)

        gathers = [None] * chunks
        outs = [None] * chunks
        gathers[0] = pltpu.async_copy(tbl_hbm.at[ridx_v.at[0]],
                                      raw_v.at[0], gsem)
        for j in range(chunks):
            if j + 1 < chunks:
                gathers[j + 1] = pltpu.async_copy(
                    tbl_hbm.at[ridx_v.at[j + 1]], raw_v.at[(j + 1) % 2], gsem)
            gathers[j].wait()
            if j >= 2:
                outs[j - 2].wait()
            extract_chunk(j, j % 2)
            outs[j] = pltpu.async_copy(
                outb_v.at[j % 2],
                out_hbm.at[pl.ds(base + j * _CHUNK, _CHUNK)], osem)
        outs[chunks - 2].wait()
        outs[chunks - 1].wait()

    return gather_kernel(feat3d, tbl128)


_VCHUNK = 512
_NFULL = _VOCAB // _VCHUNK
_VTAIL = _VOCAB - _NFULL * _VCHUNK
_FULL_STEPS = _N_FIELDS * _NFULL


def _sc_transpose(tv2, tail):
    info = plsc.get_sparse_core_info()
    nc = info.num_cores
    nw = nc * info.num_subcores
    per_tile = (_FULL_STEPS + nw - 1) // nw
    rows_full = _VCHUNK // 4
    rows_tail = _VTAIL // 4
    vpf = _VOCAB // 4

    mesh = plsc.VectorSubcoreMesh(core_axis_name="c", subcore_axis_name="s")

    @functools.partial(
        pl.kernel,
        mesh=mesh,
        out_type=jax.ShapeDtypeStruct((_N_FIELDS * vpf, 128), jnp.float32),
        scratch_types=[
            pltpu.VMEM((2, _EMBED, _VCHUNK), jnp.float32),
            pltpu.VMEM((2, _EMBED, _VCHUNK + 48), jnp.float32),
            pltpu.VMEM((2, rows_full, 128), jnp.float32),
            pltpu.SemaphoreType.DMA,
            pltpu.SemaphoreType.DMA,
            pltpu.SemaphoreType.DMA,
            pltpu.SemaphoreType.DMA,
        ],
        compiler_params=pltpu.CompilerParams(use_tc_tiling_on_sc=True,
                                             needs_layout_passes=False),
    )
    def transpose_kernel(tv_hbm, tail_hbm, out_hbm, in_v, skew_v, outb_v,
                         isem0, isem1, osem0, osem1):
        isems = (isem0, isem1)
        osems = (osem0, osem1)
        wid = lax.axis_index("s") * nc + lax.axis_index("c")
        lo = wid * per_tile
        hi = jnp.minimum(lo + per_tile, _FULL_STEPS)
        iota = lax.iota(jnp.int32, 16)

        def fetch(g, buf):
            f = g // _NFULL
            s = g % _NFULL
            pltpu.async_copy(
                tv_hbm.at[pl.ds(f * _EMBED, _EMBED),
                          pl.ds(s * _VCHUNK, _VCHUNK)],
                in_v.at[buf], isems[buf])

        def wait_in(buf):
            pltpu.make_async_copy(
                tv_hbm.at[pl.ds(0, _EMBED), pl.ds(0, _VCHUNK)],
                in_v.at[buf], isems[buf]).wait()

        def relane(buf, nrows):
            def skew_row(d, _):
                def cp(k, _):
                    skew_v[buf, d, pl.ds(k * 16 + d, 16)] = \
                        in_v[buf, d, pl.ds(k * 16, 16)]
                    return 0
                lax.fori_loop(0, _VCHUNK // 16, cp, 0)
                return 0
            lax.fori_loop(0, _EMBED, skew_row, 0)

            def row_body(i, _):
                for k in range(8):
                    d_vec = (k % 2) * 16 + iota
                    lane = i * 4 + (k >> 1) + d_vec
                    vals = plsc.load_gather(
                        skew_v, [jnp.full((16,), buf, jnp.int32), d_vec,
                                 lane])
                    outb_v[buf, i, pl.ds(k * 16, 16)] = vals
                return 0
            lax.fori_loop(0, nrows, row_body, 0)

        def writeout(g, buf):
            f = g // _NFULL
            s = g % _NFULL
            base = f * vpf + s * rows_full
            pltpu.async_copy(outb_v.at[buf],
                             out_hbm.at[pl.ds(base, rows_full)], osems[buf])

        def wait_out(buf):
            pltpu.make_async_copy(outb_v.at[buf],
                                  out_hbm.at[pl.ds(0, rows_full)],
                                  osems[buf]).wait()

        npairs = (per_tile + 1) // 2

        @pl.when(lo < hi)
        def _():
            fetch(lo, 0)

            def pair_body(p, _):
                g0 = lo + p * 2

                @pl.when(g0 < hi)
                def _():
                    @pl.when(g0 + 1 < hi)
                    def _():
                        fetch(g0 + 1, 1)
                    wait_in(0)

                    @pl.when(p >= 1)
                    def _():
                        wait_out(0)
                    relane(0, rows_full)
                    writeout(g0, 0)

                g1 = g0 + 1

                @pl.when(g1 < hi)
                def _():
                    @pl.when(g1 + 1 < hi)
                    def _():
                        fetch(g1 + 1, 0)
                    wait_in(1)

                    @pl.when(p >= 1)
                    def _():
                        wait_out(1)
                    relane(1, rows_full)
                    writeout(g1, 1)
                return 0

            lax.fori_loop(0, npairs, pair_body, 0)
            wait_out(0)

            @pl.when(hi - lo >= 2)
            def _():
                wait_out(1)

        @pl.when(wid < _N_FIELDS)
        def _():
            pltpu.sync_copy(
                tv_hbm.at[pl.ds(wid * _EMBED, _EMBED),
                          pl.ds(_NFULL * _VCHUNK, 128)],
                in_v.at[0, :, pl.ds(0, 128)])
            relane(0, 32)
            pltpu.sync_copy(
                outb_v.at[0, pl.ds(0, 32)],
                out_hbm.at[pl.ds(wid * vpf + _NFULL * rows_full, 32)])
            pltpu.sync_copy(tail_hbm.at[wid], outb_v.at[1, pl.ds(0, 8)])
            pltpu.sync_copy(
                outb_v.at[1, pl.ds(0, 8)],
                out_hbm.at[pl.ds(wid * vpf + _NFULL * rows_full + 32, 8)])

    return transpose_kernel(tv2, tail)


def _tc_mlp(x, gamma, beta, w1, b1, w2, b2, w3, b3):
    def body(x_ref, g_ref, be_ref, w1_ref, b1_ref, w2_ref, b2_ref, w3_ref,
             b3_ref, o_ref):
        xv = x_ref[...]
        inv_n = 1.0 / xv.shape[0]
        mean = jnp.sum(xv, axis=0, keepdims=True) * inv_n
        ex2 = jnp.sum(xv * xv, axis=0, keepdims=True) * inv_n
        var = ex2 - mean * mean
        scale = g_ref[...] * lax.rsqrt(var + 1e-5)
        shift = be_ref[...] - mean * scale
        xn = xv * scale + shift
        h = lax.dot_general(xn, w1_ref[...], (((1,), (1,)), ((), ())),
                            preferred_element_type=jnp.float32)
        h = jnp.maximum(h + b1_ref[...], 0.0)
        h = lax.dot_general(h, w2_ref[...], (((1,), (1,)), ((), ())),
                            preferred_element_type=jnp.float32)
        h = jnp.maximum(h + b2_ref[...], 0.0)
        logits = lax.dot_general(h, w3_ref[...], (((1,), (1,)), ((), ())),
                                 preferred_element_type=jnp.float32)
        o_ref[...] = jax.nn.sigmoid(logits + b3_ref[0])

    n_in = 9
    w3_pad = jnp.zeros((8, w3.shape[1]), w3.dtype).at[0].set(w3[0])
    out = pl.pallas_call(
        body,
        out_shape=jax.ShapeDtypeStruct((_BATCH, 8), jnp.float32),
        in_specs=[
            pl.BlockSpec(memory_space=pltpu.SMEM) if i == n_in - 1
            else pl.BlockSpec(memory_space=pltpu.VMEM)
            for i in range(n_in)
        ],
    )(x, gamma.reshape(1, -1), beta.reshape(1, -1), w1, b1.reshape(1, -1),
      w2, b2.reshape(1, -1), w3_pad, b3)
    return out[:, 0]


def kernel(features, tables, gamma, beta, w1, b1, w2, b2, w3, b3):
    feat3d = features.reshape(32, _N_FIELDS, _CHUNK)
    tv2 = jnp.transpose(tables, (0, 2, 1)).reshape(_N_FIELDS * _EMBED, _VOCAB)
    tail = tables[:, _VOCAB - 32:, :].reshape(_N_FIELDS, 8, 128)
    tbl128 = _sc_transpose(tv2, tail)
    rows = _sc_gather(feat3d, tbl128)
    x = rows.reshape(_BATCH, _N_FIELDS * _EMBED)
    out = _tc_mlp(x, gamma, beta, w1, b1, w2, b2, w3, b3)
    return out.reshape(_BATCH)


# speedup vs baseline: 1.7499x; 1.2474x over previous
import functools

import jax
import jax.numpy as jnp
from jax import lax
from jax.experimental import pallas as pl
from jax.experimental.pallas import tpu as pltpu
from jax.experimental.pallas import tpu_sc as plsc

_N_FIELDS = 26
_VOCAB = 100000
_EMBED = 32
_BATCH = 4096
_FLAT = _BATCH * _N_FIELDS
_CHUNK = 128


def _sc_gather(feat_flat, tables_flat):
    info = plsc.get_sparse_core_info()
    nc, ns = info.num_cores, info.num_subcores
    nw = nc * ns
    per_tile = _FLAT // nw
    chunks = per_tile // _CHUNK
    nvecs = per_tile // 16

    mesh = plsc.VectorSubcoreMesh(core_axis_name="c", subcore_axis_name="s")

    @functools.partial(
        pl.kernel,
        mesh=mesh,
        out_type=jax.ShapeDtypeStruct((_FLAT, _EMBED), jnp.float32),
        scratch_types=[
            pltpu.VMEM((per_tile,), jnp.int32),
            pltpu.VMEM((per_tile, _EMBED), jnp.float32),
            pltpu.SemaphoreType.DMA,
        ],
        compiler_params=pltpu.CompilerParams(use_tc_tiling_on_sc=False),
    )
    def gather_kernel(feat_hbm, tbl_hbm, out_hbm, idx_v, rows_v, sem):
        wid = lax.axis_index("s") * nc + lax.axis_index("c")
        base = wid * per_tile
        pltpu.sync_copy(feat_hbm.at[pl.ds(base, per_tile)], idx_v)

        def vec_body(t, _):
            v = idx_v[pl.ds(t * 16, 16)]
            v = jnp.clip(v, 0, _VOCAB - 1)
            pos = t * 16 + lax.iota(jnp.int32, 16)
            idx_v[pl.ds(t * 16, 16)] = v + (pos % _N_FIELDS) * _VOCAB
            return 0

        lax.fori_loop(0, nvecs, vec_body, 0)

        copies = [
            pltpu.async_copy(
                tbl_hbm.at[idx_v.at[pl.ds(j * _CHUNK, _CHUNK)]],
                rows_v.at[pl.ds(j * _CHUNK, _CHUNK)],
                sem,
            )
            for j in range(chunks)
        ]
        for c in copies:
            c.wait()

        pltpu.sync_copy(rows_v, out_hbm.at[pl.ds(base, per_tile)])

    return gather_kernel(feat_flat, tables_flat)


def _tc_mlp(x, gamma, beta, w1, b1, w2, b2, w3, b3):
    def body(x_ref, g_ref, be_ref, w1_ref, b1_ref, w2_ref, b2_ref, w3_ref,
             b3_ref, o_ref):
        xv = x_ref[...]
        inv_n = 1.0 / xv.shape[0]
        mean = jnp.sum(xv, axis=0, keepdims=True) * inv_n
        ex2 = jnp.sum(xv * xv, axis=0, keepdims=True) * inv_n
        var = ex2 - mean * mean
        scale = g_ref[...] * lax.rsqrt(var + 1e-5)
        shift = be_ref[...] - mean * scale
        xn = xv * scale + shift
        h = lax.dot_general(xn, w1_ref[...], (((1,), (1,)), ((), ())),
                            preferred_element_type=jnp.float32)
        h = jnp.maximum(h + b1_ref[...], 0.0)
        h = lax.dot_general(h, w2_ref[...], (((1,), (1,)), ((), ())),
                            preferred_element_type=jnp.float32)
        h = jnp.maximum(h + b2_ref[...], 0.0)
        logits = lax.dot_general(h, w3_ref[...], (((1,), (1,)), ((), ())),
                                 preferred_element_type=jnp.float32)
        o_ref[...] = jax.nn.sigmoid(logits + b3_ref[0])

    n_in = 9
    w3_pad = jnp.zeros((8, w3.shape[1]), w3.dtype).at[0].set(w3[0])
    out = pl.pallas_call(
        body,
        out_shape=jax.ShapeDtypeStruct((_BATCH, 8), jnp.float32),
        in_specs=[
            pl.BlockSpec(memory_space=pltpu.SMEM) if i == n_in - 1
            else pl.BlockSpec(memory_space=pltpu.VMEM)
            for i in range(n_in)
        ],
    )(x, gamma.reshape(1, -1), beta.reshape(1, -1), w1, b1.reshape(1, -1),
      w2, b2.reshape(1, -1), w3_pad, b3)
    return out[:, 0]


def kernel(features, tables, gamma, beta, w1, b1, w2, b2, w3, b3):
    feat_flat = features.reshape(_FLAT)
    tables_flat = tables.reshape(_N_FIELDS * _VOCAB, _EMBED)
    rows = _sc_gather(feat_flat, tables_flat)
    x = rows.reshape(_BATCH, _N_FIELDS * _EMBED)
    out = _tc_mlp(x, gamma, beta, w1, b1, w2, b2, w3, b3)
    return out.reshape(_BATCH)
